# Initial kernel scaffold; baseline (speedup 1.0000x reference)
import jax
import jax.numpy as jnp
from jax.experimental import pallas as pl


def kernel(global_info, x, ptr):
    raise NotImplementedError("write your pallas kernel here")


# speedup vs baseline: 1.0695x; 1.0695x over previous
import functools

import jax
import jax.numpy as jnp
from jax import lax
from jax.experimental import pallas as pl
from jax.experimental.pallas import tpu as pltpu
from jax.experimental.pallas import tpu_sc as plsc

NC = 2
NS = 16
L = 16
NW = NC * NS


def kernel(global_info, x, ptr):
    B, D = global_info.shape
    TOTAL = x.shape[0]
    rows_per_w = TOTAL // NW
    C = L
    nchunks = rows_per_w // C

    mesh = plsc.VectorSubcoreMesh(core_axis_name="c", subcore_axis_name="s")

    @functools.partial(
        pl.kernel,
        out_type=jax.ShapeDtypeStruct((TOTAL, 2 * D), jnp.float32),
        mesh=mesh,
        scratch_types=[
            pltpu.VMEM((L,), jnp.int32),
            pltpu.VMEM((C, D), jnp.float32),
            pltpu.VMEM((C, D), jnp.float32),
            pltpu.VMEM((C, D), jnp.float32),
            pltpu.VMEM((C, D), jnp.float32),
            pltpu.SemaphoreType.DMA,
            pltpu.SemaphoreType.DMA,
            pltpu.SemaphoreType.DMA,
            pltpu.SemaphoreType.DMA,
        ],
    )
    def run(g_hbm, x_hbm, ptr_hbm, out_hbm, ptr_v, bx0, bx1, bg0, bg1,
            si0, si1, so0, so1):
        bx = (bx0, bx1)
        bg = (bg0, bg1)
        si = (si0, si1)
        so = (so0, so1)
        wid = lax.axis_index("s") * NC + lax.axis_index("c")
        base = wid * rows_per_w

        pltpu.sync_copy(ptr_hbm.at[pl.ds(0, L)], ptr_v)
        pv = ptr_v[...]
        ones = jnp.full((L,), 1, jnp.int32)
        zeros = jnp.zeros((L,), jnp.int32)
        pbs = [
            pv.at[jnp.full((L,), b, jnp.int32)].get(mode="promise_in_bounds")
            for b in range(1, B)
        ]

        def seg_of(row0):
            rows = row0 + lax.iota(jnp.int32, L)
            seg = zeros
            for pb in pbs:
                seg = seg + jnp.where(pb <= rows, ones, zeros)
            return seg

        def start_in(k, j):
            row0 = base + k * C
            pltpu.async_copy(x_hbm.at[pl.ds(row0, C)], bx[j], si[j])
            pltpu.async_copy(g_hbm.at[seg_of(row0)], bg[j], si[j])

        def wait_in(j):
            pltpu.make_async_copy(x_hbm.at[pl.ds(0, C)], bx[j], si[j]).wait()
            pltpu.make_async_copy(x_hbm.at[pl.ds(0, C)], bg[j], si[j]).wait()

        def start_out(k, j):
            row0 = base + k * C
            pltpu.async_copy(bx[j], out_hbm.at[pl.ds(row0, C), pl.ds(0, D)],
                             so[j])
            pltpu.async_copy(bg[j], out_hbm.at[pl.ds(row0, C), pl.ds(D, D)],
                             so[j])

        def wait_out(j):
            pltpu.make_async_copy(bx[j], out_hbm.at[pl.ds(0, C), pl.ds(0, D)],
                                  so[j]).wait()
            pltpu.make_async_copy(bg[j], out_hbm.at[pl.ds(0, C), pl.ds(D, D)],
                                  so[j]).wait()

        start_in(0, 0)
        start_in(1, 1)
        wait_in(0)
        start_out(0, 0)
        wait_in(1)
        start_out(1, 1)
        wait_out(0)
        start_in(2, 0)

        def step(k2, carry):
            for j in range(2):
                k = 2 * k2 + j
                wait_in(j)
                start_out(k, j)
                wait_out(j ^ 1)

                @pl.when(k < nchunks - 1)
                def _():
                    start_in(k + 1, j ^ 1)
            return carry

        lax.fori_loop(1, nchunks // 2, step, 0)
        wait_out(1)

    return run(global_info, x, ptr)
